# baseline (speedup 1.0000x reference)
import functools

import jax
import jax.numpy as jnp
from jax import lax
from jax.experimental import pallas as pl
from jax.experimental.pallas import tpu as pltpu
from jax.experimental.pallas import tpu_sc as plsc

NC = 2
NS = 16
NW = NC * NS
LANES = 16


EC = 128


GR = 8


def _zero_spmem(buf, sh, n, d, sid, cid_unused=None):
    nfull = n // EC
    ntail = n - nfull * EC
    kmax = (nfull + NS - 1) // NS
    for k in range(kmax):
        idx = k * NS + sid
        @pl.when(idx < nfull)
        def _():
            pltpu.sync_copy(buf, sh.at[pl.ds(idx * EC, EC)])
    if ntail:
        @pl.when(sid == NS - 1)
        def _():
            pltpu.sync_copy(buf.at[pl.ds(0, ntail)], sh.at[pl.ds(nfull * EC, ntail)])


def _dump_spmem(buf, sh, out, n, d, sid, cid):
    nfull = n // EC
    ntail = n - nfull * EC
    kmax = (nfull + NS - 1) // NS
    for k in range(kmax):
        idx = k * NS + sid
        @pl.when(idx < nfull)
        def _():
            r0 = idx * EC
            pltpu.sync_copy(sh.at[pl.ds(r0, EC)], buf)
            pltpu.sync_copy(buf, out.at[cid, pl.ds(r0, EC)])
    if ntail:
        @pl.when(sid == NS - 1)
        def _():
            r0 = nfull * EC
            pltpu.sync_copy(sh.at[pl.ds(r0, ntail)], buf.at[pl.ds(0, ntail)])
            pltpu.sync_copy(buf.at[pl.ds(0, ntail)], out.at[cid, pl.ds(r0, ntail)])


@functools.partial(jax.jit, static_argnames=("n", "d", "nch"))
def _agg_call(h, src3d, dst3d, *, n, d, nch):
    ngr = nch // GR
    assert ngr * GR == nch

    mesh = plsc.VectorSubcoreMesh(core_axis_name="c", subcore_axis_name="s")

    scratch = [
        pltpu.VMEM((3, GR, EC), jnp.int32),
        pltpu.VMEM((3, GR, EC), jnp.int32),
        pltpu.VMEM((2, EC, d), jnp.float32),
        pltpu.VMEM_SHARED((n + LANES, d), jnp.float32),
        pltpu.SemaphoreType.DMA,
        pltpu.SemaphoreType.DMA,
        pltpu.SemaphoreType.DMA,
        pltpu.SemaphoreType.DMA,
        pltpu.SemaphoreType.DMA,
        pltpu.SemaphoreType.DMA,
        pltpu.SemaphoreType.DMA,
    ]

    def body(h_hbm, src_hbm, dst_hbm, agg_out,
             src_v, dst_v, rows_v, agg_sh, si0, si1, si2, sg0, sg1, ss0, ss1):
        si = [si0, si1, si2]
        sg = [sg0, sg1]
        ss = [ss0, ss1]
        cid = lax.axis_index("c")
        sid = lax.axis_index("s")
        wid = sid * NC + cid

        zeros = jnp.zeros((LANES,), jnp.float32)
        zbuf = rows_v.at[0]

        def zfill(i, _):
            for s in range(d // LANES):
                zbuf[i, pl.ds(s * LANES, LANES)] = zeros
            return 0
        lax.fori_loop(0, EC, zfill, 0)
        _zero_spmem(zbuf, agg_sh, n, d, sid)
        plsc.subcore_barrier()

        def issue_idx(g):
            gb = g % 3
            pltpu.async_copy(src_hbm.at[wid, pl.ds(g * GR, GR)], src_v.at[gb], si[gb])
            pltpu.async_copy(dst_hbm.at[wid, pl.ds(g * GR, GR)], dst_v.at[gb], si[gb])

        def wait_idx(g):
            gb = g % 3
            pltpu.make_async_copy(src_hbm.at[wid, pl.ds(g * GR, GR)], src_v.at[gb], si[gb]).wait()
            pltpu.make_async_copy(dst_hbm.at[wid, pl.ds(g * GR, GR)], dst_v.at[gb], si[gb]).wait()

        def _kgb(j):
            if isinstance(j, int):
                return j % GR, (j // GR) % 3
            return lax.rem(j, GR), lax.rem(lax.div(j, GR), 3)

        def issue_gather(j, b):
            k, gb = _kgb(j)
            pltpu.async_copy(h_hbm.at[src_v.at[gb, k]], rows_v.at[b], sg[b])

        def wait_gather(j, b):
            k, gb = _kgb(j)
            pltpu.make_async_copy(h_hbm.at[src_v.at[gb, k]], rows_v.at[b], sg[b]).wait()

        def issue_scatter(j, b):
            k, gb = _kgb(j)
            pltpu.async_copy(rows_v.at[b], agg_sh.at[dst_v.at[gb, k]], ss[b], add=True)

        def wait_scatter(j, b):
            k, gb = _kgb(j)
            pltpu.make_async_copy(rows_v.at[b], agg_sh.at[dst_v.at[gb, k]], ss[b]).wait()

        issue_idx(0)
        wait_idx(0)
        issue_gather(0, 0)
        npair = GR // 2
        for g in range(ngr):
            if g + 1 < ngr:
                issue_idx(g + 1)

            def pair_body(u, _, g=g):
                j0 = g * GR + 2 * u
                wait_gather(j0, 0)
                issue_scatter(j0, 0)

                @pl.when(j0 > 0)
                def _():
                    wait_scatter(j0 - 1, 1)
                issue_gather(j0 + 1, 1)
                wait_gather(j0 + 1, 1)
                issue_scatter(j0 + 1, 1)
                wait_scatter(j0, 0)

                @pl.when(u < npair - 1)
                def _():
                    issue_gather(j0 + 2, 0)
                return 0
            lax.fori_loop(0, npair, pair_body, 0)
            if g + 1 < ngr:
                wait_idx(g + 1)
                issue_gather((g + 1) * GR, 0)
        wait_scatter(nch - 1, 1)

        plsc.subcore_barrier()
        _dump_spmem(zbuf, agg_sh, agg_out, n, d, sid, cid)

    fn = pl.kernel(body,
                   out_type=jax.ShapeDtypeStruct((NC, n, d), jnp.float32),
                   mesh=mesh, scratch_types=scratch)
    return fn(h, src3d, dst3d)


@functools.partial(jax.jit, static_argnames=("n", "d", "nch"))
def _deg_call(dst3d, *, n, d, nch):
    ngr = nch // GR
    assert ngr * GR == nch

    mesh = plsc.VectorSubcoreMesh(core_axis_name="c", subcore_axis_name="s")

    scratch = [
        pltpu.VMEM((3, GR, EC), jnp.int32),
        pltpu.VMEM((EC, d), jnp.float32),
        pltpu.VMEM_SHARED((n + LANES, d), jnp.float32),
        pltpu.SemaphoreType.DMA,
        pltpu.SemaphoreType.DMA,
        pltpu.SemaphoreType.DMA,
        pltpu.SemaphoreType.DMA,
        pltpu.SemaphoreType.DMA,
        pltpu.SemaphoreType.DMA,
        pltpu.SemaphoreType.DMA,
    ]

    def body(dst_hbm, deg_out, dst_v, ones_v, deg_sh, si0, si1, si2, s0, s1, s2, s3):
        si = [si0, si1, si2]
        ssem = [s0, s1, s2, s3]
        cid = lax.axis_index("c")
        sid = lax.axis_index("s")
        wid = sid * NC + cid

        zeros = jnp.zeros((LANES,), jnp.float32)

        def zfill(i, _):
            for s in range(d // LANES):
                ones_v[i, pl.ds(s * LANES, LANES)] = zeros
            return 0
        lax.fori_loop(0, EC, zfill, 0)
        _zero_spmem(ones_v, deg_sh, n, d, sid)

        def ofill(i, _):
            for s in range(d // LANES):
                ones_v[i, pl.ds(s * LANES, LANES)] = zeros + 1.0
            return 0
        lax.fori_loop(0, EC, ofill, 0)
        plsc.subcore_barrier()

        def issue_idx(g):
            gb = g % 3
            pltpu.async_copy(dst_hbm.at[wid, pl.ds(g * GR, GR)], dst_v.at[gb], si[gb])

        def wait_idx(g):
            gb = g % 3
            pltpu.make_async_copy(dst_hbm.at[wid, pl.ds(g * GR, GR)], dst_v.at[gb], si[gb]).wait()

        def _kgb(j):
            if isinstance(j, int):
                return j % GR, (j // GR) % 3
            return lax.rem(j, GR), lax.rem(lax.div(j, GR), 3)

        def issue_sc(j, q):
            k, gb = _kgb(j)
            pltpu.async_copy(ones_v, deg_sh.at[dst_v.at[gb, k]], ssem[q], add=True)

        def wait_sc(j, q):
            k, gb = _kgb(j)
            pltpu.make_async_copy(ones_v, deg_sh.at[dst_v.at[gb, k]], ssem[q]).wait()

        issue_idx(0)
        wait_idx(0)
        for g in range(ngr):
            if g + 1 < ngr:
                issue_idx(g + 1)

            def qbody(u, _, g=g):
                base = g * GR + 4 * u
                for q in range(4):
                    j = base + q
                    @pl.when(j >= 4)
                    def _():
                        wait_sc(j - 4, q)
                    issue_sc(j, q)
                return 0
            lax.fori_loop(0, GR // 4, qbody, 0)
            if g + 1 < ngr:
                wait_idx(g + 1)
        for q in range(4):
            wait_sc(nch - 4 + q, q)

        plsc.subcore_barrier()
        _dump_spmem(ones_v, deg_sh, deg_out, n, d, sid, cid)

    fn = pl.kernel(body,
                   out_type=jax.ShapeDtypeStruct((NC, n, d), jnp.float32),
                   mesh=mesh, scratch_types=scratch)
    return fn(dst3d)


def _layer_body(h_ref, aggp_ref, degp_ref, ws_ref, wn_ref, b_ref, out_ref):
    agg = aggp_ref[0] + aggp_ref[1]
    deg = degp_ref[0, :, 0] + degp_ref[1, :, 0]
    invd = 1.0 / jnp.maximum(deg, 1.0)
    m = agg * invd[:, None]
    z = (jnp.dot(h_ref[...], ws_ref[...], preferred_element_type=jnp.float32)
         + jnp.dot(m, wn_ref[...], preferred_element_type=jnp.float32)
         + b_ref[...])
    z = jnp.where(z >= 0, z, 0.2 * z)
    nrm = jnp.sqrt(jnp.sum(z * z, axis=-1, keepdims=True)) + 1e-12
    out_ref[...] = z / nrm


@functools.partial(jax.jit, static_argnames=("n", "d", "bn"))
def _layer_call(h, aggp, degp, ws, wn, b, *, n, d, bn):
    grid = (n // bn,)
    return pl.pallas_call(
        _layer_body,
        grid=grid,
        in_specs=[
            pl.BlockSpec((bn, d), lambda i: (i, 0)),
            pl.BlockSpec((NC, bn, d), lambda i: (0, i, 0)),
            pl.BlockSpec((NC, bn, d), lambda i: (0, i, 0)),
            pl.BlockSpec((d, d), lambda i: (0, 0)),
            pl.BlockSpec((d, d), lambda i: (0, 0)),
            pl.BlockSpec((1, d), lambda i: (0, 0)),
        ],
        out_specs=pl.BlockSpec((bn, d), lambda i: (i, 0)),
        out_shape=jax.ShapeDtypeStruct((n, d), jnp.float32),
    )(h, aggp, degp, ws, wn, b.reshape(1, d))


@functools.partial(jax.jit, static_argnames=("n", "d", "ptot"))
def _scores_call(h, a2d, b2d, *, n, d, ptot):
    CS = 128
    ppw = ptot // NW
    nch = ppw // CS
    assert ppw * NW == ptot and nch * CS == ppw

    mesh = plsc.VectorSubcoreMesh(core_axis_name="c", subcore_axis_name="s")

    def body(h_hbm, a_hbm, b_hbm, out_hbm, a_v, b_v, ra_v, rb_v, out_v,
             sa0, sa1, sb0, sb1):
        sa = [sa0, sa1]
        sb = [sb0, sb1]
        cid = lax.axis_index("c")
        sid = lax.axis_index("s")
        wid = sid * NC + cid

        pltpu.sync_copy(a_hbm.at[wid], a_v)
        pltpu.sync_copy(b_hbm.at[wid], b_v)

        def issue(j, b):
            pltpu.async_copy(h_hbm.at[a_v.at[j]], ra_v.at[b], sa[b])
            pltpu.async_copy(h_hbm.at[b_v.at[j]], rb_v.at[b], sb[b])

        def wait(j, b):
            pltpu.make_async_copy(h_hbm.at[a_v.at[j]], ra_v.at[b], sa[b]).wait()
            pltpu.make_async_copy(h_hbm.at[b_v.at[j]], rb_v.at[b], sb[b]).wait()

        def compute(j, b):
            ra = ra_v.at[b]
            rb = rb_v.at[b]

            def pair(p, _):
                acc = ra[p, pl.ds(0, LANES)] * rb[p, pl.ds(0, LANES)]
                for s in range(1, d // LANES):
                    acc = acc + (ra[p, pl.ds(s * LANES, LANES)]
                                 * rb[p, pl.ds(s * LANES, LANES)])
                out_v[p, pl.ds(0, LANES)] = acc
                return 0
            lax.fori_loop(0, CS, pair, 0, unroll=4)
            pltpu.sync_copy(out_v, out_hbm.at[pl.ds((wid * nch + j) * CS, CS)])

        issue(0, 0)

        def chunk_pair(t, _):
            j0 = 2 * t
            issue(j0 + 1, 1)
            wait(j0, 0)
            compute(j0, 0)

            @pl.when(j0 + 2 < nch)
            def _():
                issue(j0 + 2, 0)
            wait(j0 + 1, 1)
            compute(j0 + 1, 1)
            return 0
        lax.fori_loop(0, nch // 2, chunk_pair, 0)

    fn = pl.kernel(
        body,
        out_type=jax.ShapeDtypeStruct((ptot, LANES), jnp.float32),
        mesh=mesh,
        scratch_types=[
            pltpu.VMEM((nch, CS), jnp.int32),
            pltpu.VMEM((nch, CS), jnp.int32),
            pltpu.VMEM((2, CS, d), jnp.float32),
            pltpu.VMEM((2, CS, d), jnp.float32),
            pltpu.VMEM((CS, LANES), jnp.float32),
            pltpu.SemaphoreType.DMA,
            pltpu.SemaphoreType.DMA,
            pltpu.SemaphoreType.DMA,
            pltpu.SemaphoreType.DMA,
        ],
    )
    return fn(h, a2d, b2d)


def _score_reduce_body(part_ref, out_ref):
    out_ref[...] = jnp.sum(part_ref[...], axis=-1)


@functools.partial(jax.jit, static_argnames=("ptot", "bp"))
def _score_reduce(part, *, ptot, bp):
    return pl.pallas_call(
        _score_reduce_body,
        grid=(ptot // bp,),
        in_specs=[pl.BlockSpec((bp, LANES), lambda i: (i, 0))],
        out_specs=pl.BlockSpec((bp,), lambda i: (i,)),
        out_shape=jax.ShapeDtypeStruct((ptot,), jnp.float32),
    )(part)


def kernel(x, edge_index, pos_edge_index, neg_edge_index,
           W_self_0, W_neigh_0, b_0,
           W_self_1, W_neigh_1, b_1,
           W_self_2, W_neigh_2, b_2):
    n, d = x.shape
    e = edge_index.shape[1]
    npair = pos_edge_index.shape[1]

    equant = NW * EC * GR
    epad = ((e + equant - 1) // equant) * equant
    nch = epad // (NW * EC)
    src_p = jnp.concatenate([edge_index[0], jnp.zeros((epad - e,), jnp.int32)])
    dst_p = jnp.concatenate([edge_index[1], jnp.full((epad - e,), n, jnp.int32)])
    src3d = src_p.reshape(NW, nch, EC)
    dst3d = dst_p.reshape(NW, nch, EC)

    ws = [(W_self_0, W_neigh_0, b_0), (W_self_1, W_neigh_1, b_1), (W_self_2, W_neigh_2, b_2)]
    degp = _deg_call(dst3d, n=n, d=d, nch=nch)
    h = x
    for (wself, wneigh, bb) in ws:
        aggp = _agg_call(h, src3d, dst3d, n=n, d=d, nch=nch)
        h = _layer_call(h, aggp, degp, wself, wneigh, bb, n=n, d=d, bn=1000)
    h3 = h

    CS = 128
    quant = NW * CS
    npad = ((npair + quant - 1) // quant) * quant
    pz = npad - npair

    def padcat(row):
        return jnp.concatenate([row, jnp.zeros((pz,), jnp.int32)])

    a_all = jnp.concatenate([padcat(pos_edge_index[0]), padcat(neg_edge_index[0])])
    b_all = jnp.concatenate([padcat(pos_edge_index[1]), padcat(neg_edge_index[1])])
    nchs = 2 * npad // (NW * CS)
    part = _scores_call(h3, a_all.reshape(NW, nchs, CS), b_all.reshape(NW, nchs, CS),
                        n=n, d=d, ptot=2 * npad)
    scores = _score_reduce(part, ptot=2 * npad, bp=2048)
    pos_score = scores[:npair]
    neg_score = scores[npad:npad + npair]
    return (h3, pos_score, neg_score)


# speedup vs baseline: 3.0110x; 3.0110x over previous
import functools

import jax
import jax.numpy as jnp
from jax import lax
from jax.experimental import pallas as pl
from jax.experimental.pallas import tpu as pltpu
from jax.experimental.pallas import tpu_sc as plsc

NC = 2
NS = 16
NW = NC * NS
LANES = 16


EC = 128


GR = 8
TRASH = 128


def _zero_spmem(buf, sh, n, d, sid, cid_unused=None):
    nfull = n // EC
    ntail = n - nfull * EC
    kmax = (nfull + NS - 1) // NS
    for k in range(kmax):
        idx = k * NS + sid
        @pl.when(idx < nfull)
        def _():
            pltpu.sync_copy(buf, sh.at[pl.ds(idx * EC, EC)])
    if ntail:
        @pl.when(sid == NS - 1)
        def _():
            pltpu.sync_copy(buf.at[pl.ds(0, ntail)], sh.at[pl.ds(nfull * EC, ntail)])


def _dump_spmem(buf, sh, out, n, d, sid, cid):
    nfull = n // EC
    ntail = n - nfull * EC
    kmax = (nfull + NS - 1) // NS
    for k in range(kmax):
        idx = k * NS + sid
        @pl.when(idx < nfull)
        def _():
            r0 = idx * EC
            pltpu.sync_copy(sh.at[pl.ds(r0, EC)], buf)
            pltpu.sync_copy(buf, out.at[cid, pl.ds(r0, EC)])
    if ntail:
        @pl.when(sid == NS - 1)
        def _():
            r0 = nfull * EC
            pltpu.sync_copy(sh.at[pl.ds(r0, ntail)], buf.at[pl.ds(0, ntail)])
            pltpu.sync_copy(buf.at[pl.ds(0, ntail)], out.at[cid, pl.ds(r0, ntail)])


@functools.partial(jax.jit, static_argnames=("n", "d", "nch"))
def _agg_call(h, src3d, dst3d, *, n, d, nch):
    ngr = nch // GR
    assert ngr * GR == nch

    mesh = plsc.VectorSubcoreMesh(core_axis_name="c", subcore_axis_name="s")

    scratch = [
        pltpu.VMEM((3, GR, EC), jnp.int32),
        pltpu.VMEM((3, GR, EC), jnp.int32),
        pltpu.VMEM((2, EC, d), jnp.float32),
        pltpu.VMEM_SHARED((n + TRASH, d), jnp.float32),
        pltpu.SemaphoreType.DMA,
        pltpu.SemaphoreType.DMA,
        pltpu.SemaphoreType.DMA,
        pltpu.SemaphoreType.DMA,
        pltpu.SemaphoreType.DMA,
        pltpu.SemaphoreType.DMA,
        pltpu.SemaphoreType.DMA,
    ]

    def body(h_hbm, src_hbm, dst_hbm, agg_out,
             src_v, dst_v, rows_v, agg_sh, si0, si1, si2, sg0, sg1, ss0, ss1):
        si = [si0, si1, si2]
        sg = [sg0, sg1]
        ss = [ss0, ss1]
        cid = lax.axis_index("c")
        sid = lax.axis_index("s")
        wid = sid * NC + cid

        zeros = jnp.zeros((LANES,), jnp.float32)
        zbuf = rows_v.at[0]

        def zfill(i, _):
            for s in range(d // LANES):
                zbuf[i, pl.ds(s * LANES, LANES)] = zeros
            return 0
        lax.fori_loop(0, EC, zfill, 0)
        _zero_spmem(zbuf, agg_sh, n, d, sid)
        plsc.subcore_barrier()

        def issue_idx(g):
            gb = g % 3
            pltpu.async_copy(src_hbm.at[wid, pl.ds(g * GR, GR)], src_v.at[gb], si[gb])
            pltpu.async_copy(dst_hbm.at[wid, pl.ds(g * GR, GR)], dst_v.at[gb], si[gb])

        def wait_idx(g):
            gb = g % 3
            pltpu.make_async_copy(src_hbm.at[wid, pl.ds(g * GR, GR)], src_v.at[gb], si[gb]).wait()
            pltpu.make_async_copy(dst_hbm.at[wid, pl.ds(g * GR, GR)], dst_v.at[gb], si[gb]).wait()

        def _kgb(j):
            if isinstance(j, int):
                return j % GR, (j // GR) % 3
            return lax.rem(j, GR), lax.rem(lax.div(j, GR), 3)

        def issue_gather(j, b):
            k, gb = _kgb(j)
            pltpu.async_copy(h_hbm.at[src_v.at[gb, k]], rows_v.at[b], sg[b])

        def wait_gather(j, b):
            k, gb = _kgb(j)
            pltpu.make_async_copy(h_hbm.at[src_v.at[gb, k]], rows_v.at[b], sg[b]).wait()

        def issue_scatter(j, b):
            k, gb = _kgb(j)
            pltpu.async_copy(rows_v.at[b], agg_sh.at[dst_v.at[gb, k]], ss[b], add=True)

        def wait_scatter(j, b):
            k, gb = _kgb(j)
            pltpu.make_async_copy(rows_v.at[b], agg_sh.at[dst_v.at[gb, k]], ss[b]).wait()

        issue_idx(0)
        wait_idx(0)
        issue_gather(0, 0)
        npair = GR // 2
        for g in range(ngr):
            if g + 1 < ngr:
                issue_idx(g + 1)

            def pair_body(u, _, g=g):
                j0 = g * GR + 2 * u
                wait_gather(j0, 0)
                issue_scatter(j0, 0)

                @pl.when(j0 > 0)
                def _():
                    wait_scatter(j0 - 1, 1)
                issue_gather(j0 + 1, 1)
                wait_gather(j0 + 1, 1)
                issue_scatter(j0 + 1, 1)
                wait_scatter(j0, 0)

                @pl.when(u < npair - 1)
                def _():
                    issue_gather(j0 + 2, 0)
                return 0
            lax.fori_loop(0, npair, pair_body, 0)
            if g + 1 < ngr:
                wait_idx(g + 1)
                issue_gather((g + 1) * GR, 0)
        wait_scatter(nch - 1, 1)

        plsc.subcore_barrier()
        _dump_spmem(zbuf, agg_sh, agg_out, n, d, sid, cid)

    fn = pl.kernel(body,
                   out_type=jax.ShapeDtypeStruct((NC, n, d), jnp.float32),
                   mesh=mesh, scratch_types=scratch)
    return fn(h, src3d, dst3d)


@functools.partial(jax.jit, static_argnames=("n", "d", "nch"))
def _deg_call(dst3d, *, n, d, nch):
    ngr = nch // GR
    assert ngr * GR == nch

    mesh = plsc.VectorSubcoreMesh(core_axis_name="c", subcore_axis_name="s")

    scratch = [
        pltpu.VMEM((3, GR, EC), jnp.int32),
        pltpu.VMEM((EC, d), jnp.float32),
        pltpu.VMEM_SHARED((n + TRASH, d), jnp.float32),
        pltpu.SemaphoreType.DMA,
        pltpu.SemaphoreType.DMA,
        pltpu.SemaphoreType.DMA,
        pltpu.SemaphoreType.DMA,
        pltpu.SemaphoreType.DMA,
        pltpu.SemaphoreType.DMA,
        pltpu.SemaphoreType.DMA,
    ]

    def body(dst_hbm, deg_out, dst_v, ones_v, deg_sh, si0, si1, si2, s0, s1, s2, s3):
        si = [si0, si1, si2]
        ssem = [s0, s1, s2, s3]
        cid = lax.axis_index("c")
        sid = lax.axis_index("s")
        wid = sid * NC + cid

        zeros = jnp.zeros((LANES,), jnp.float32)

        def zfill(i, _):
            for s in range(d // LANES):
                ones_v[i, pl.ds(s * LANES, LANES)] = zeros
            return 0
        lax.fori_loop(0, EC, zfill, 0)
        _zero_spmem(ones_v, deg_sh, n, d, sid)

        def ofill(i, _):
            for s in range(d // LANES):
                ones_v[i, pl.ds(s * LANES, LANES)] = zeros + 1.0
            return 0
        lax.fori_loop(0, EC, ofill, 0)
        plsc.subcore_barrier()

        def issue_idx(g):
            gb = g % 3
            pltpu.async_copy(dst_hbm.at[wid, pl.ds(g * GR, GR)], dst_v.at[gb], si[gb])

        def wait_idx(g):
            gb = g % 3
            pltpu.make_async_copy(dst_hbm.at[wid, pl.ds(g * GR, GR)], dst_v.at[gb], si[gb]).wait()

        def _kgb(j):
            if isinstance(j, int):
                return j % GR, (j // GR) % 3
            return lax.rem(j, GR), lax.rem(lax.div(j, GR), 3)

        def issue_sc(j, q):
            k, gb = _kgb(j)
            pltpu.async_copy(ones_v, deg_sh.at[dst_v.at[gb, k]], ssem[q], add=True)

        def wait_sc(j, q):
            k, gb = _kgb(j)
            pltpu.make_async_copy(ones_v, deg_sh.at[dst_v.at[gb, k]], ssem[q]).wait()

        issue_idx(0)
        wait_idx(0)
        for g in range(ngr):
            if g + 1 < ngr:
                issue_idx(g + 1)

            def qbody(u, _, g=g):
                base = g * GR + 4 * u
                for q in range(4):
                    j = base + q
                    @pl.when(j >= 4)
                    def _():
                        wait_sc(j - 4, q)
                    issue_sc(j, q)
                return 0
            lax.fori_loop(0, GR // 4, qbody, 0)
            if g + 1 < ngr:
                wait_idx(g + 1)
        for q in range(4):
            wait_sc(nch - 4 + q, q)

        plsc.subcore_barrier()
        _dump_spmem(ones_v, deg_sh, deg_out, n, d, sid, cid)

    fn = pl.kernel(body,
                   out_type=jax.ShapeDtypeStruct((NC, n, d), jnp.float32),
                   mesh=mesh, scratch_types=scratch)
    return fn(dst3d)


def _layer_body(h_ref, aggp_ref, degp_ref, ws_ref, wn_ref, b_ref, out_ref):
    agg = aggp_ref[0] + aggp_ref[1]
    deg = degp_ref[0, :, 0] + degp_ref[1, :, 0]
    invd = 1.0 / jnp.maximum(deg, 1.0)
    m = agg * invd[:, None]
    z = (jnp.dot(h_ref[...], ws_ref[...], preferred_element_type=jnp.float32)
         + jnp.dot(m, wn_ref[...], preferred_element_type=jnp.float32)
         + b_ref[...])
    z = jnp.where(z >= 0, z, 0.2 * z)
    nrm = jnp.sqrt(jnp.sum(z * z, axis=-1, keepdims=True)) + 1e-12
    out_ref[...] = z / nrm


@functools.partial(jax.jit, static_argnames=("n", "d", "bn"))
def _layer_call(h, aggp, degp, ws, wn, b, *, n, d, bn):
    grid = (n // bn,)
    return pl.pallas_call(
        _layer_body,
        grid=grid,
        in_specs=[
            pl.BlockSpec((bn, d), lambda i: (i, 0)),
            pl.BlockSpec((NC, bn, d), lambda i: (0, i, 0)),
            pl.BlockSpec((NC, bn, d), lambda i: (0, i, 0)),
            pl.BlockSpec((d, d), lambda i: (0, 0)),
            pl.BlockSpec((d, d), lambda i: (0, 0)),
            pl.BlockSpec((1, d), lambda i: (0, 0)),
        ],
        out_specs=pl.BlockSpec((bn, d), lambda i: (i, 0)),
        out_shape=jax.ShapeDtypeStruct((n, d), jnp.float32),
    )(h, aggp, degp, ws, wn, b.reshape(1, d))


@functools.partial(jax.jit, static_argnames=("n", "d", "ptot"))
def _scores_call(h, a2d, b2d, *, n, d, ptot):
    CS = 128
    ppw = ptot // NW
    nch = ppw // CS
    assert ppw * NW == ptot and nch * CS == ppw

    mesh = plsc.VectorSubcoreMesh(core_axis_name="c", subcore_axis_name="s")

    def body(h_hbm, a_hbm, b_hbm, out_hbm, a_v, b_v, ra_v, rb_v, out_v,
             sa0, sa1, sb0, sb1):
        sa = [sa0, sa1]
        sb = [sb0, sb1]
        cid = lax.axis_index("c")
        sid = lax.axis_index("s")
        wid = sid * NC + cid

        pltpu.sync_copy(a_hbm.at[wid], a_v)
        pltpu.sync_copy(b_hbm.at[wid], b_v)

        def issue(j, b):
            pltpu.async_copy(h_hbm.at[a_v.at[j]], ra_v.at[b], sa[b])
            pltpu.async_copy(h_hbm.at[b_v.at[j]], rb_v.at[b], sb[b])

        def wait(j, b):
            pltpu.make_async_copy(h_hbm.at[a_v.at[j]], ra_v.at[b], sa[b]).wait()
            pltpu.make_async_copy(h_hbm.at[b_v.at[j]], rb_v.at[b], sb[b]).wait()

        def compute(j, b):
            ra = ra_v.at[b]
            rb = rb_v.at[b]

            def pair(p, _):
                acc = ra[p, pl.ds(0, LANES)] * rb[p, pl.ds(0, LANES)]
                for s in range(1, d // LANES):
                    acc = acc + (ra[p, pl.ds(s * LANES, LANES)]
                                 * rb[p, pl.ds(s * LANES, LANES)])
                out_v[p, pl.ds(0, LANES)] = acc
                return 0
            lax.fori_loop(0, CS, pair, 0, unroll=4)
            pltpu.sync_copy(out_v, out_hbm.at[pl.ds((wid * nch + j) * CS, CS)])

        issue(0, 0)

        def chunk_pair(t, _):
            j0 = 2 * t
            issue(j0 + 1, 1)
            wait(j0, 0)
            compute(j0, 0)

            @pl.when(j0 + 2 < nch)
            def _():
                issue(j0 + 2, 0)
            wait(j0 + 1, 1)
            compute(j0 + 1, 1)
            return 0
        lax.fori_loop(0, nch // 2, chunk_pair, 0)

    fn = pl.kernel(
        body,
        out_type=jax.ShapeDtypeStruct((ptot, LANES), jnp.float32),
        mesh=mesh,
        scratch_types=[
            pltpu.VMEM((nch, CS), jnp.int32),
            pltpu.VMEM((nch, CS), jnp.int32),
            pltpu.VMEM((2, CS, d), jnp.float32),
            pltpu.VMEM((2, CS, d), jnp.float32),
            pltpu.VMEM((CS, LANES), jnp.float32),
            pltpu.SemaphoreType.DMA,
            pltpu.SemaphoreType.DMA,
            pltpu.SemaphoreType.DMA,
            pltpu.SemaphoreType.DMA,
        ],
    )
    return fn(h, a2d, b2d)


def _score_reduce_body(part_ref, out_ref):
    out_ref[...] = jnp.sum(part_ref[...], axis=-1)


@functools.partial(jax.jit, static_argnames=("ptot", "bp"))
def _score_reduce(part, *, ptot, bp):
    return pl.pallas_call(
        _score_reduce_body,
        grid=(ptot // bp,),
        in_specs=[pl.BlockSpec((bp, LANES), lambda i: (i, 0))],
        out_specs=pl.BlockSpec((bp,), lambda i: (i,)),
        out_shape=jax.ShapeDtypeStruct((ptot,), jnp.float32),
    )(part)


def kernel(x, edge_index, pos_edge_index, neg_edge_index,
           W_self_0, W_neigh_0, b_0,
           W_self_1, W_neigh_1, b_1,
           W_self_2, W_neigh_2, b_2):
    n, d = x.shape
    e = edge_index.shape[1]
    npair = pos_edge_index.shape[1]

    equant = NW * EC * GR
    epad = ((e + equant - 1) // equant) * equant
    nch = epad // (NW * EC)
    pidx = jnp.arange(epad - e, dtype=jnp.int32)
    src_p = jnp.concatenate([edge_index[0], pidx % n])
    dst_p = jnp.concatenate([edge_index[1], n + pidx % TRASH])
    src3d = src_p.reshape(NW, nch, EC)
    dst3d = dst_p.reshape(NW, nch, EC)

    ws = [(W_self_0, W_neigh_0, b_0), (W_self_1, W_neigh_1, b_1), (W_self_2, W_neigh_2, b_2)]
    degp = _deg_call(dst3d, n=n, d=d, nch=nch)
    h = x
    for (wself, wneigh, bb) in ws:
        aggp = _agg_call(h, src3d, dst3d, n=n, d=d, nch=nch)
        h = _layer_call(h, aggp, degp, wself, wneigh, bb, n=n, d=d, bn=1000)
    h3 = h

    CS = 128
    quant = NW * CS
    npad = ((npair + quant - 1) // quant) * quant
    pz = npad - npair

    sidx = jnp.arange(pz, dtype=jnp.int32)

    def padcat(row):
        return jnp.concatenate([row, sidx % n])

    a_all = jnp.concatenate([padcat(pos_edge_index[0]), padcat(neg_edge_index[0])])
    b_all = jnp.concatenate([padcat(pos_edge_index[1]), padcat(neg_edge_index[1])])
    nchs = 2 * npad // (NW * CS)
    part = _scores_call(h3, a_all.reshape(NW, nchs, CS), b_all.reshape(NW, nchs, CS),
                        n=n, d=d, ptot=2 * npad)
    scores = _score_reduce(part, ptot=2 * npad, bp=2048)
    pos_score = scores[:npair]
    neg_score = scores[npad:npad + npair]
    return (h3, pos_score, neg_score)


# speedup vs baseline: 3.4324x; 1.1399x over previous
import functools

import jax
import jax.numpy as jnp
from jax import lax
from jax.experimental import pallas as pl
from jax.experimental.pallas import tpu as pltpu
from jax.experimental.pallas import tpu_sc as plsc

NC = 2
NS = 16
NW = NC * NS
LANES = 16


EC = 128


GR = 8
TRASH = 128


def _zero_spmem(buf, sh, n, d, sid, cid_unused=None):
    nfull = n // EC
    ntail = n - nfull * EC
    kmax = (nfull + NS - 1) // NS
    for k in range(kmax):
        idx = k * NS + sid
        @pl.when(idx < nfull)
        def _():
            pltpu.sync_copy(buf, sh.at[pl.ds(idx * EC, EC)])
    if ntail:
        @pl.when(sid == NS - 1)
        def _():
            pltpu.sync_copy(buf.at[pl.ds(0, ntail)], sh.at[pl.ds(nfull * EC, ntail)])


def _dump_spmem(buf, sh, out, n, d, sid, cid):
    nfull = n // EC
    ntail = n - nfull * EC
    kmax = (nfull + NS - 1) // NS
    for k in range(kmax):
        idx = k * NS + sid
        @pl.when(idx < nfull)
        def _():
            r0 = idx * EC
            pltpu.sync_copy(sh.at[pl.ds(r0, EC)], buf)
            pltpu.sync_copy(buf, out.at[cid, pl.ds(r0, EC)])
    if ntail:
        @pl.when(sid == NS - 1)
        def _():
            r0 = nfull * EC
            pltpu.sync_copy(sh.at[pl.ds(r0, ntail)], buf.at[pl.ds(0, ntail)])
            pltpu.sync_copy(buf.at[pl.ds(0, ntail)], out.at[cid, pl.ds(r0, ntail)])


@functools.partial(jax.jit, static_argnames=("n", "d", "nch"))
def _agg_call(h, src3d, dst3d, *, n, d, nch):
    ngr = nch // GR
    assert ngr * GR == nch

    mesh = plsc.VectorSubcoreMesh(core_axis_name="c", subcore_axis_name="s")

    scratch = [
        pltpu.VMEM((3, GR, EC), jnp.int32),
        pltpu.VMEM((3, GR, EC), jnp.int32),
        pltpu.VMEM((2, EC, d), jnp.float32),
        pltpu.VMEM_SHARED((n + TRASH, d), jnp.float32),
        pltpu.SemaphoreType.DMA,
        pltpu.SemaphoreType.DMA,
        pltpu.SemaphoreType.DMA,
        pltpu.SemaphoreType.DMA,
        pltpu.SemaphoreType.DMA,
        pltpu.SemaphoreType.DMA,
        pltpu.SemaphoreType.DMA,
    ]

    def body(h_hbm, src_hbm, dst_hbm, agg_out,
             src_v, dst_v, rows_v, agg_sh, si0, si1, si2, sg0, sg1, ss0, ss1):
        si = [si0, si1, si2]
        sg = [sg0, sg1]
        ss = [ss0, ss1]
        cid = lax.axis_index("c")
        sid = lax.axis_index("s")
        wid = sid * NC + cid

        zeros = jnp.zeros((LANES,), jnp.float32)
        zbuf = rows_v.at[0]

        def zfill(i, _):
            for s in range(d // LANES):
                zbuf[i, pl.ds(s * LANES, LANES)] = zeros
            return 0
        lax.fori_loop(0, EC, zfill, 0)
        _zero_spmem(zbuf, agg_sh, n, d, sid)
        plsc.subcore_barrier()

        def issue_idx(g):
            gb = g % 3
            pltpu.async_copy(src_hbm.at[wid, pl.ds(g * GR, GR)], src_v.at[gb], si[gb])
            pltpu.async_copy(dst_hbm.at[wid, pl.ds(g * GR, GR)], dst_v.at[gb], si[gb])

        def wait_idx(g):
            gb = g % 3
            pltpu.make_async_copy(src_hbm.at[wid, pl.ds(g * GR, GR)], src_v.at[gb], si[gb]).wait()
            pltpu.make_async_copy(dst_hbm.at[wid, pl.ds(g * GR, GR)], dst_v.at[gb], si[gb]).wait()

        def _kgb(j):
            if isinstance(j, int):
                return j % GR, (j // GR) % 3
            return lax.rem(j, GR), lax.rem(lax.div(j, GR), 3)

        def issue_gather(j, b):
            k, gb = _kgb(j)
            pltpu.async_copy(h_hbm.at[src_v.at[gb, k]], rows_v.at[b], sg[b])

        def wait_gather(j, b):
            k, gb = _kgb(j)
            pltpu.make_async_copy(h_hbm.at[src_v.at[gb, k]], rows_v.at[b], sg[b]).wait()

        def issue_scatter(j, b):
            k, gb = _kgb(j)
            pltpu.async_copy(rows_v.at[b], agg_sh.at[dst_v.at[gb, k]], ss[b], add=True)

        def wait_scatter(j, b):
            k, gb = _kgb(j)
            pltpu.make_async_copy(rows_v.at[b], agg_sh.at[dst_v.at[gb, k]], ss[b]).wait()

        issue_idx(0)
        wait_idx(0)
        issue_gather(0, 0)
        npair = GR // 2
        for g in range(ngr):
            if g + 1 < ngr:
                issue_idx(g + 1)

            def pair_body(u, _, g=g):
                j0 = g * GR + 2 * u
                wait_gather(j0, 0)
                issue_scatter(j0, 0)

                @pl.when(j0 > 0)
                def _():
                    wait_scatter(j0 - 1, 1)
                issue_gather(j0 + 1, 1)
                wait_gather(j0 + 1, 1)
                issue_scatter(j0 + 1, 1)
                wait_scatter(j0, 0)

                @pl.when(u < npair - 1)
                def _():
                    issue_gather(j0 + 2, 0)
                return 0
            lax.fori_loop(0, npair, pair_body, 0)
            if g + 1 < ngr:
                wait_idx(g + 1)
                issue_gather((g + 1) * GR, 0)
        wait_scatter(nch - 1, 1)

        plsc.subcore_barrier()
        _dump_spmem(zbuf, agg_sh, agg_out, n, d, sid, cid)

    fn = pl.kernel(body,
                   out_type=jax.ShapeDtypeStruct((NC, n, d), jnp.float32),
                   mesh=mesh, scratch_types=scratch)
    return fn(h, src3d, dst3d)


@functools.partial(jax.jit, static_argnames=("n", "d", "nch"))
def _deg_call(dst3d, *, n, d, nch):
    ngr = nch // GR
    assert ngr * GR == nch

    mesh = plsc.VectorSubcoreMesh(core_axis_name="c", subcore_axis_name="s")

    scratch = [
        pltpu.VMEM((3, GR, EC), jnp.int32),
        pltpu.VMEM((EC, d), jnp.float32),
        pltpu.VMEM_SHARED((n + TRASH, d), jnp.float32),
        pltpu.SemaphoreType.DMA,
        pltpu.SemaphoreType.DMA,
        pltpu.SemaphoreType.DMA,
        pltpu.SemaphoreType.DMA,
        pltpu.SemaphoreType.DMA,
        pltpu.SemaphoreType.DMA,
        pltpu.SemaphoreType.DMA,
    ]

    def body(dst_hbm, deg_out, dst_v, ones_v, deg_sh, si0, si1, si2, s0, s1, s2, s3):
        si = [si0, si1, si2]
        ssem = [s0, s1, s2, s3]
        cid = lax.axis_index("c")
        sid = lax.axis_index("s")
        wid = sid * NC + cid

        zeros = jnp.zeros((LANES,), jnp.float32)

        def zfill(i, _):
            for s in range(d // LANES):
                ones_v[i, pl.ds(s * LANES, LANES)] = zeros
            return 0
        lax.fori_loop(0, EC, zfill, 0)
        _zero_spmem(ones_v, deg_sh, n, d, sid)

        def ofill(i, _):
            for s in range(d // LANES):
                ones_v[i, pl.ds(s * LANES, LANES)] = zeros + 1.0
            return 0
        lax.fori_loop(0, EC, ofill, 0)
        plsc.subcore_barrier()

        def issue_idx(g):
            gb = g % 3
            pltpu.async_copy(dst_hbm.at[wid, pl.ds(g * GR, GR)], dst_v.at[gb], si[gb])

        def wait_idx(g):
            gb = g % 3
            pltpu.make_async_copy(dst_hbm.at[wid, pl.ds(g * GR, GR)], dst_v.at[gb], si[gb]).wait()

        def _kgb(j):
            if isinstance(j, int):
                return j % GR, (j // GR) % 3
            return lax.rem(j, GR), lax.rem(lax.div(j, GR), 3)

        def issue_sc(j, q):
            k, gb = _kgb(j)
            pltpu.async_copy(ones_v, deg_sh.at[dst_v.at[gb, k]], ssem[q], add=True)

        def wait_sc(j, q):
            k, gb = _kgb(j)
            pltpu.make_async_copy(ones_v, deg_sh.at[dst_v.at[gb, k]], ssem[q]).wait()

        issue_idx(0)
        wait_idx(0)
        for g in range(ngr):
            if g + 1 < ngr:
                issue_idx(g + 1)

            def qbody(u, _, g=g):
                base = g * GR + 4 * u
                for q in range(4):
                    j = base + q
                    @pl.when(j >= 4)
                    def _():
                        wait_sc(j - 4, q)
                    issue_sc(j, q)
                return 0
            lax.fori_loop(0, GR // 4, qbody, 0)
            if g + 1 < ngr:
                wait_idx(g + 1)
        for q in range(4):
            wait_sc(nch - 4 + q, q)

        plsc.subcore_barrier()
        _dump_spmem(ones_v, deg_sh, deg_out, n, d, sid, cid)

    fn = pl.kernel(body,
                   out_type=jax.ShapeDtypeStruct((NC, n, d), jnp.float32),
                   mesh=mesh, scratch_types=scratch)
    return fn(dst3d)


def _layer_body(h_ref, aggp_ref, degp_ref, ws_ref, wn_ref, b_ref, out_ref):
    agg = aggp_ref[0] + aggp_ref[1]
    deg = degp_ref[0, :, 0] + degp_ref[1, :, 0]
    invd = 1.0 / jnp.maximum(deg, 1.0)
    m = agg * invd[:, None]
    z = (jnp.dot(h_ref[...], ws_ref[...], preferred_element_type=jnp.float32)
         + jnp.dot(m, wn_ref[...], preferred_element_type=jnp.float32)
         + b_ref[...])
    z = jnp.where(z >= 0, z, 0.2 * z)
    nrm = jnp.sqrt(jnp.sum(z * z, axis=-1, keepdims=True)) + 1e-12
    out_ref[...] = z / nrm


@functools.partial(jax.jit, static_argnames=("n", "d", "bn"))
def _layer_call(h, aggp, degp, ws, wn, b, *, n, d, bn):
    grid = (n // bn,)
    return pl.pallas_call(
        _layer_body,
        grid=grid,
        in_specs=[
            pl.BlockSpec((bn, d), lambda i: (i, 0)),
            pl.BlockSpec((NC, bn, d), lambda i: (0, i, 0)),
            pl.BlockSpec((NC, bn, d), lambda i: (0, i, 0)),
            pl.BlockSpec((d, d), lambda i: (0, 0)),
            pl.BlockSpec((d, d), lambda i: (0, 0)),
            pl.BlockSpec((1, d), lambda i: (0, 0)),
        ],
        out_specs=pl.BlockSpec((bn, d), lambda i: (i, 0)),
        out_shape=jax.ShapeDtypeStruct((n, d), jnp.float32),
    )(h, aggp, degp, ws, wn, b.reshape(1, d))


@functools.partial(jax.jit, static_argnames=("n", "d", "ptot"))
def _scores_call(h, a2d, b2d, *, n, d, ptot):
    CS = 128
    ppw = ptot // NW
    nch = ppw // CS
    assert ppw * NW == ptot and nch * CS == ppw

    mesh = plsc.VectorSubcoreMesh(core_axis_name="c", subcore_axis_name="s")

    def body(h_hbm, a_hbm, b_hbm, out_hbm, a_v, b_v, ra_v, rb_v, out_v,
             sa0, sa1, sb0, sb1):
        sa = [sa0, sa1]
        sb = [sb0, sb1]
        cid = lax.axis_index("c")
        sid = lax.axis_index("s")
        wid = sid * NC + cid

        pltpu.sync_copy(a_hbm.at[wid], a_v)
        pltpu.sync_copy(b_hbm.at[wid], b_v)

        def issue(j, b):
            pltpu.async_copy(h_hbm.at[a_v.at[j]], ra_v.at[b], sa[b])
            pltpu.async_copy(h_hbm.at[b_v.at[j]], rb_v.at[b], sb[b])

        def wait(j, b):
            pltpu.make_async_copy(h_hbm.at[a_v.at[j]], ra_v.at[b], sa[b]).wait()
            pltpu.make_async_copy(h_hbm.at[b_v.at[j]], rb_v.at[b], sb[b]).wait()

        def compute(j, b):
            ra = ra_v.at[b]
            rb = rb_v.at[b]

            def row8(r, _):
                for q in range(8):
                    p = r * 8 + q
                    acc = ra[p, pl.ds(0, LANES)] * rb[p, pl.ds(0, LANES)]
                    for s in range(1, d // LANES):
                        acc = acc + (ra[p, pl.ds(s * LANES, LANES)]
                                     * rb[p, pl.ds(s * LANES, LANES)])
                    out_v[r, pl.ds(q * LANES, LANES)] = acc
                return 0
            lax.fori_loop(0, CS // 8, row8, 0, unroll=2)
            pltpu.sync_copy(out_v, out_hbm.at[pl.ds((wid * nch + j) * (CS // 8), CS // 8)])

        issue(0, 0)

        def chunk_pair(t, _):
            j0 = 2 * t
            issue(j0 + 1, 1)
            wait(j0, 0)
            compute(j0, 0)

            @pl.when(j0 + 2 < nch)
            def _():
                issue(j0 + 2, 0)
            wait(j0 + 1, 1)
            compute(j0 + 1, 1)
            return 0
        lax.fori_loop(0, nch // 2, chunk_pair, 0)

    fn = pl.kernel(
        body,
        out_type=jax.ShapeDtypeStruct((ptot // 8, d), jnp.float32),
        mesh=mesh,
        scratch_types=[
            pltpu.VMEM((nch, CS), jnp.int32),
            pltpu.VMEM((nch, CS), jnp.int32),
            pltpu.VMEM((2, CS, d), jnp.float32),
            pltpu.VMEM((2, CS, d), jnp.float32),
            pltpu.VMEM((CS // 8, d), jnp.float32),
            pltpu.SemaphoreType.DMA,
            pltpu.SemaphoreType.DMA,
            pltpu.SemaphoreType.DMA,
            pltpu.SemaphoreType.DMA,
        ],
    )
    return fn(h, a2d, b2d)


def _score_reduce_body(part_ref, out_ref):
    d = part_ref.shape[1]
    ng = d // LANES
    l = lax.broadcasted_iota(jnp.int32, (d, ng), 0)
    g = lax.broadcasted_iota(jnp.int32, (d, ng), 1)
    m = (l // LANES == g).astype(jnp.float32)
    out_ref[...] = jnp.dot(part_ref[...], m, preferred_element_type=jnp.float32)


@functools.partial(jax.jit, static_argnames=("ptot", "bp", "d"))
def _score_reduce(part, *, ptot, bp, d):
    rows = ptot // 8
    return pl.pallas_call(
        _score_reduce_body,
        grid=(rows // bp,),
        in_specs=[pl.BlockSpec((bp, d), lambda i: (i, 0))],
        out_specs=pl.BlockSpec((bp, d // LANES), lambda i: (i, 0)),
        out_shape=jax.ShapeDtypeStruct((rows, d // LANES), jnp.float32),
    )(part)


def kernel(x, edge_index, pos_edge_index, neg_edge_index,
           W_self_0, W_neigh_0, b_0,
           W_self_1, W_neigh_1, b_1,
           W_self_2, W_neigh_2, b_2):
    n, d = x.shape
    e = edge_index.shape[1]
    npair = pos_edge_index.shape[1]

    equant = NW * EC * GR
    epad = ((e + equant - 1) // equant) * equant
    nch = epad // (NW * EC)
    pidx = jnp.arange(epad - e, dtype=jnp.int32)
    src_p = jnp.concatenate([edge_index[0], pidx % n])
    dst_p = jnp.concatenate([edge_index[1], n + pidx % TRASH])
    src3d = src_p.reshape(NW, nch, EC)
    dst3d = dst_p.reshape(NW, nch, EC)

    ws = [(W_self_0, W_neigh_0, b_0), (W_self_1, W_neigh_1, b_1), (W_self_2, W_neigh_2, b_2)]
    degp = _deg_call(dst3d, n=n, d=d, nch=nch)
    h = x
    for (wself, wneigh, bb) in ws:
        aggp = _agg_call(h, src3d, dst3d, n=n, d=d, nch=nch)
        h = _layer_call(h, aggp, degp, wself, wneigh, bb, n=n, d=d, bn=1000)
    h3 = h

    CS = 128
    quant = NW * CS
    npad = ((npair + quant - 1) // quant) * quant
    pz = npad - npair

    sidx = jnp.arange(pz, dtype=jnp.int32)

    def padcat(row):
        return jnp.concatenate([row, sidx % n])

    a_all = jnp.concatenate([padcat(pos_edge_index[0]), padcat(neg_edge_index[0])])
    b_all = jnp.concatenate([padcat(pos_edge_index[1]), padcat(neg_edge_index[1])])
    nchs = 2 * npad // (NW * CS)
    part = _scores_call(h3, a_all.reshape(NW, nchs, CS), b_all.reshape(NW, nchs, CS),
                        n=n, d=d, ptot=2 * npad)
    scores = _score_reduce(part, ptot=2 * npad, bp=1600, d=d).reshape(-1)
    pos_score = scores[:npair]
    neg_score = scores[npad:npad + npair]
    return (h3, pos_score, neg_score)
